# Initial kernel scaffold; baseline (speedup 1.0000x reference)
import jax
import jax.numpy as jnp
from jax.experimental import pallas as pl


def kernel(tokens, cu_seqlens):
    raise NotImplementedError("write your pallas kernel here")


# speedup vs baseline: 10.0739x; 10.0739x over previous
import functools

import jax
import jax.numpy as jnp
from jax import lax
from jax.experimental import pallas as pl
from jax.experimental.pallas import tpu as pltpu
from jax.experimental.pallas import tpu_sc as plsc

SEQ = 2048
HALF = 1024
TOK = 16384
FRONT = 8
PAD_LEN = 18448
WIN = HALF + 16


def _sc_pack(padded_hbm, starts_hbm, ends_hbm, out_hbm, sv, ev, buf, obuf):
    row = lax.axis_index("s")
    h = lax.axis_index("c")
    w = row * 2 + h
    pltpu.sync_copy(starts_hbm, sv)
    pltpu.sync_copy(ends_hbm, ev)
    iota16 = lax.broadcasted_iota(jnp.int32, (16,), 0)
    start = sv[pl.ds(row, 16)][0]
    seg = ev[pl.ds(row, 16)][0] - start
    jbase = h * HALF
    lo = start + (FRONT - 1) + jbase
    b = pl.multiple_of((lo // 8) * 8, 8)
    d = lo - b
    pltpu.sync_copy(padded_hbm.at[pl.ds(b, WIN)], buf)

    def body(c, carry):
        jv = jbase + c * 16 + iota16
        vals = buf[pl.ds(d + c * 16, 16)]
        r = jnp.where(
            jv == 0,
            jnp.float32(1.0),
            jnp.where(
                jv <= seg,
                vals,
                jnp.where(jv == seg + 1, jnp.float32(2.0), jnp.float32(0.0)),
            ),
        )
        obuf[pl.ds(c * 16, 16)] = r
        return carry

    lax.fori_loop(0, HALF // 16, body, 0)
    pltpu.sync_copy(obuf, out_hbm.at[w])


def kernel(tokens, cu_seqlens):
    padded = jnp.concatenate([
        jnp.zeros((FRONT,), jnp.float32),
        tokens,
        jnp.zeros((PAD_LEN - FRONT - TOK,), jnp.float32),
    ])
    starts = jnp.concatenate([cu_seqlens[:16], jnp.zeros((16,), jnp.int32)])
    ends = jnp.concatenate([cu_seqlens[1:17], jnp.zeros((16,), jnp.int32)])
    mesh = plsc.VectorSubcoreMesh(core_axis_name="c", subcore_axis_name="s")
    f = pl.kernel(
        _sc_pack,
        mesh=mesh,
        out_type=jax.ShapeDtypeStruct((32, HALF), jnp.float32),
        scratch_types=[
            pltpu.VMEM((32,), jnp.int32),
            pltpu.VMEM((32,), jnp.int32),
            pltpu.VMEM((WIN,), jnp.float32),
            pltpu.VMEM((HALF,), jnp.float32),
        ],
    )
    return f(padded, starts, ends).reshape(16, SEQ)


# speedup vs baseline: 11.1863x; 1.1104x over previous
import jax
import jax.numpy as jnp
from jax import lax
from jax.experimental import pallas as pl
from jax.experimental.pallas import tpu as pltpu
from jax.experimental.pallas import tpu_sc as plsc

SEQ = 2048
HALF = 1024
TOK = 16384
WIN = HALF + 16
BUF = WIN + 24
BMAX = TOK - WIN


def _sc_pack(tok_hbm, cu_hbm, out_hbm, cuv, buf, obuf):
    row = lax.axis_index("s")
    h = lax.axis_index("c")
    w = row * 2 + h
    pltpu.sync_copy(cu_hbm.at[pl.ds(0, 16)], cuv.at[pl.ds(0, 16)])
    v = cuv[pl.ds(row, 16)]
    start = v[0]
    end = jnp.where(row == 15, TOK, v[1])
    seg = end - start
    jbase = h * HALF
    b = jnp.clip(((start + jbase - 1) // 8) * 8, 0, BMAX)
    b = pl.multiple_of(b, 8)
    d8 = start + jbase + 7 - b
    pltpu.sync_copy(tok_hbm.at[pl.ds(b, WIN)], buf.at[pl.ds(8, WIN)])
    iota16 = lax.broadcasted_iota(jnp.int32, (16,), 0)
    nc = jnp.clip((seg + 17 - jbase) // 16, 0, HALF // 16)

    def content(c, carry):
        jv = jbase + c * 16 + iota16
        vals = buf[pl.ds(d8 + c * 16, 16)]
        r = jnp.where(
            jv == 0,
            jnp.float32(1.0),
            jnp.where(
                jv <= seg,
                vals,
                jnp.where(jv == seg + 1, jnp.float32(2.0), jnp.float32(0.0)),
            ),
        )
        obuf[pl.ds(c * 16, 16)] = r
        return carry

    def padfill(c, carry):
        obuf[pl.ds(c * 16, 16)] = jnp.zeros((16,), jnp.float32)
        return carry

    lax.fori_loop(0, nc, content, 0)
    lax.fori_loop(nc, HALF // 16, padfill, 0)
    pltpu.sync_copy(obuf, out_hbm.at[w])


def kernel(tokens, cu_seqlens):
    mesh = plsc.VectorSubcoreMesh(core_axis_name="c", subcore_axis_name="s")
    f = pl.kernel(
        _sc_pack,
        mesh=mesh,
        out_type=jax.ShapeDtypeStruct((32, HALF), jnp.float32),
        scratch_types=[
            pltpu.VMEM((32,), jnp.int32),
            pltpu.VMEM((BUF,), jnp.float32),
            pltpu.VMEM((HALF,), jnp.float32),
        ],
    )
    return f(tokens, cu_seqlens).reshape(16, SEQ)


# speedup vs baseline: 12.1313x; 1.0845x over previous
import jax
import jax.numpy as jnp
from jax import lax
from jax.experimental import pallas as pl
from jax.experimental.pallas import tpu as pltpu
from jax.experimental.pallas import tpu_sc as plsc

SEQ = 2048
HALF = 1024
TOK = 16384
WIN = HALF + 16
BUF = WIN + 24
BMAX = TOK - WIN


def _sc_pack(tok_hbm, cu_hbm, out_hbm, cuv, buf, obuf):
    row = lax.axis_index("s")
    h = lax.axis_index("c")
    pltpu.sync_copy(cu_hbm.at[pl.ds(0, 16)], cuv.at[pl.ds(0, 16)])
    v = cuv[pl.ds(row, 16)]
    start = v[0]
    end = jnp.where(row == 15, TOK, v[1])
    seg = end - start
    jbase = h * HALF
    b = jnp.clip(((start + jbase - 1) // 8) * 8, 0, BMAX)
    b = pl.multiple_of(b, 8)
    d8 = start + jbase + 7 - b
    pltpu.sync_copy(tok_hbm.at[pl.ds(b, WIN)], buf.at[pl.ds(8, WIN)])
    iota16 = lax.broadcasted_iota(jnp.int32, (16,), 0)
    nc = jnp.clip((seg + 17 - jbase) // 16, 0, HALF // 16)

    def content(c, carry):
        jv = jbase + c * 16 + iota16
        vals = buf[pl.ds(d8 + c * 16, 16)]
        r = jnp.where(
            jv == 0,
            jnp.float32(1.0),
            jnp.where(
                jv <= seg,
                vals,
                jnp.where(jv == seg + 1, jnp.float32(2.0), jnp.float32(0.0)),
            ),
        )
        obuf[pl.ds(c * 16, 16)] = r
        return carry

    def padfill(c, carry):
        obuf[pl.ds(c * 16, 16)] = jnp.zeros((16,), jnp.float32)
        return carry

    lax.fori_loop(0, nc, content, 0)
    lax.fori_loop(nc, HALF // 16, padfill, 0)
    pltpu.sync_copy(obuf, out_hbm.at[row, pl.ds(jbase, HALF)])


def kernel(tokens, cu_seqlens):
    mesh = plsc.VectorSubcoreMesh(core_axis_name="c", subcore_axis_name="s")
    f = pl.kernel(
        _sc_pack,
        mesh=mesh,
        out_type=jax.ShapeDtypeStruct((16, SEQ), jnp.float32),
        scratch_types=[
            pltpu.VMEM((32,), jnp.int32),
            pltpu.VMEM((BUF,), jnp.float32),
            pltpu.VMEM((HALF,), jnp.float32),
        ],
    )
    return f(tokens, cu_seqlens)


# speedup vs baseline: 12.7435x; 1.0505x over previous
import jax
import jax.numpy as jnp
from jax import lax
from jax.experimental import pallas as pl
from jax.experimental.pallas import tpu as pltpu
from jax.experimental.pallas import tpu_sc as plsc

SEQ = 2048
TOK = 16384
WIN = SEQ + 16
BUF = WIN + 24
BMAX = TOK - WIN


def _sc_pack(tok_hbm, cu_hbm, out_hbm, cuv, buf, obuf):
    row = lax.axis_index("s")
    pltpu.sync_copy(cu_hbm.at[pl.ds(0, 16)], cuv.at[pl.ds(0, 16)])
    v = cuv[pl.ds(row, 16)]
    start = v[0]
    end = jnp.where(row == 15, TOK, v[1])
    seg = end - start
    b = jnp.clip(((start - 1) // 8) * 8, 0, BMAX)
    b = pl.multiple_of(b, 8)
    d8 = start + 7 - b
    pltpu.sync_copy(tok_hbm.at[pl.ds(b, WIN)], buf.at[pl.ds(8, WIN)])
    iota16 = lax.broadcasted_iota(jnp.int32, (16,), 0)
    nc = jnp.clip((seg + 17) // 16, 0, SEQ // 16)

    def content(c, carry):
        jv = c * 16 + iota16
        vals = buf[pl.ds(d8 + c * 16, 16)]
        r = jnp.where(
            jv == 0,
            jnp.float32(1.0),
            jnp.where(
                jv <= seg,
                vals,
                jnp.where(jv == seg + 1, jnp.float32(2.0), jnp.float32(0.0)),
            ),
        )
        obuf[pl.ds(c * 16, 16)] = r
        return carry

    def padfill(c, carry):
        obuf[pl.ds(c * 16, 16)] = jnp.zeros((16,), jnp.float32)
        return carry

    lax.fori_loop(0, nc, content, 0)
    lax.fori_loop(nc, SEQ // 16, padfill, 0)
    pltpu.sync_copy(obuf, out_hbm.at[row])


def kernel(tokens, cu_seqlens):
    mesh = plsc.VectorSubcoreMesh(
        core_axis_name="c", subcore_axis_name="s", num_cores=1
    )
    f = pl.kernel(
        _sc_pack,
        mesh=mesh,
        out_type=jax.ShapeDtypeStruct((16, SEQ), jnp.float32),
        scratch_types=[
            pltpu.VMEM((32,), jnp.int32),
            pltpu.VMEM((BUF,), jnp.float32),
            pltpu.VMEM((SEQ,), jnp.float32),
        ],
    )
    return f(tokens, cu_seqlens)
